# baseline (speedup 1.0000x reference)
import jax
import jax.numpy as jnp
from jax.experimental import pallas as pl
from jax.experimental.pallas import tpu as pltpu

B, C, H, W = 16, 384, 64, 64
D = H * W
HB = 8
NH = B // HB
NT = C * NH
RING = 64
LAG = 32


def _dma_body(idx_ref, x_hbm, out_hbm, buf, gsems, ssems):
    pending_in = [None] * RING
    pending_out = [None] * RING

    def start_in(t):
        c, h = t // NH, t % NH
        slot = t % RING
        d = pltpu.make_async_copy(
            x_hbm.at[pl.ds(HB * h, HB), pl.ds(idx_ref[c], 1), :],
            buf.at[slot],
            gsems.at[slot],
        )
        d.start()
        pending_in[slot] = d

    def start_out(t):
        c, h = t // NH, t % NH
        slot = t % RING
        pending_in[slot].wait()
        d = pltpu.make_async_copy(
            buf.at[slot],
            out_hbm.at[pl.ds(HB * h, HB), pl.ds(c, 1), :],
            ssems.at[slot],
        )
        d.start()
        pending_out[slot] = d

    for t in range(NT + LAG):
        if t < NT:
            slot = t % RING
            if pending_out[slot] is not None:
                pending_out[slot].wait()
                pending_out[slot] = None
            start_in(t)
        if t >= LAG:
            start_out(t - LAG)
    for slot in range(RING):
        if pending_out[slot] is not None:
            pending_out[slot].wait()


@jax.jit
def _shuffle(x3d, perm32):
    f = pl.pallas_call(
        _dma_body,
        grid_spec=pltpu.PrefetchScalarGridSpec(
            num_scalar_prefetch=1,
            grid=(1,),
            in_specs=[pl.BlockSpec(memory_space=pltpu.HBM)],
            out_specs=pl.BlockSpec(memory_space=pltpu.HBM),
            scratch_shapes=[
                pltpu.VMEM((RING, HB, 1, D), jnp.float32),
                pltpu.SemaphoreType.DMA((RING,)),
                pltpu.SemaphoreType.DMA((RING,)),
            ],
        ),
        out_shape=jax.ShapeDtypeStruct((B, C, D), jnp.float32),
    )
    return f(perm32, x3d)


def kernel(inputs, permutation):
    x3d = inputs.reshape(B, C, D)
    perm32 = permutation.astype(jnp.int32)
    return _shuffle(x3d, perm32).reshape(B, C, H, W)


# speedup vs baseline: 1.0061x; 1.0061x over previous
import jax
import jax.numpy as jnp
from jax.experimental import pallas as pl
from jax.experimental.pallas import tpu as pltpu

B, C, H, W = 16, 384, 64, 64
D = H * W
RING = 96
LAG = 48


def _dma_body(idx_ref, x_hbm, out_hbm, buf, gsems, ssems):
    pending_in = [None] * RING
    pending_out = [None] * RING

    def start_in(c):
        slot = c % RING
        d = pltpu.make_async_copy(
            x_hbm.at[:, pl.ds(idx_ref[c], 1), :],
            buf.at[slot],
            gsems.at[slot],
        )
        d.start()
        pending_in[slot] = d

    def start_out(c):
        slot = c % RING
        pending_in[slot].wait()
        d = pltpu.make_async_copy(
            buf.at[slot],
            out_hbm.at[:, pl.ds(c, 1), :],
            ssems.at[slot],
        )
        d.start()
        pending_out[slot] = d

    for c in range(C + LAG):
        if c < C:
            slot = c % RING
            if pending_out[slot] is not None:
                pending_out[slot].wait()
                pending_out[slot] = None
            start_in(c)
        if c >= LAG:
            start_out(c - LAG)
    for slot in range(RING):
        if pending_out[slot] is not None:
            pending_out[slot].wait()


@jax.jit
def _shuffle(x3d, perm32):
    f = pl.pallas_call(
        _dma_body,
        grid_spec=pltpu.PrefetchScalarGridSpec(
            num_scalar_prefetch=1,
            grid=(1,),
            in_specs=[pl.BlockSpec(memory_space=pltpu.HBM)],
            out_specs=pl.BlockSpec(memory_space=pltpu.HBM),
            scratch_shapes=[
                pltpu.VMEM((RING, B, 1, D), jnp.float32),
                pltpu.SemaphoreType.DMA((RING,)),
                pltpu.SemaphoreType.DMA((RING,)),
            ],
        ),
        out_shape=jax.ShapeDtypeStruct((B, C, D), jnp.float32),
    )
    return f(perm32, x3d)


def kernel(inputs, permutation):
    x3d = inputs.reshape(B, C, D)
    perm32 = permutation.astype(jnp.int32)
    return _shuffle(x3d, perm32).reshape(B, C, H, W)


# speedup vs baseline: 1.0083x; 1.0022x over previous
import jax
import jax.numpy as jnp
from jax.experimental import pallas as pl
from jax.experimental.pallas import tpu as pltpu

B, C, H, W = 16, 384, 64, 64
D = H * W
RING = 48
LAG = 24


def _dma_body(idx_ref, x_hbm, out_hbm, buf, gsems, ssems):
    pending_in = [None] * RING
    pending_out = [None] * RING

    def start_in(c):
        slot = c % RING
        d = pltpu.make_async_copy(
            x_hbm.at[:, pl.ds(idx_ref[c], 1), :],
            buf.at[slot],
            gsems.at[slot],
        )
        d.start()
        pending_in[slot] = d

    def start_out(c):
        slot = c % RING
        pending_in[slot].wait()
        d = pltpu.make_async_copy(
            buf.at[slot],
            out_hbm.at[:, pl.ds(c, 1), :],
            ssems.at[slot],
        )
        d.start()
        pending_out[slot] = d

    for c in range(C + LAG):
        if c < C:
            slot = c % RING
            if pending_out[slot] is not None:
                pending_out[slot].wait()
                pending_out[slot] = None
            start_in(c)
        if c >= LAG:
            start_out(c - LAG)
    for slot in range(RING):
        if pending_out[slot] is not None:
            pending_out[slot].wait()


@jax.jit
def _shuffle(x3d, perm32):
    f = pl.pallas_call(
        _dma_body,
        grid_spec=pltpu.PrefetchScalarGridSpec(
            num_scalar_prefetch=1,
            grid=(1,),
            in_specs=[pl.BlockSpec(memory_space=pltpu.HBM)],
            out_specs=pl.BlockSpec(memory_space=pltpu.HBM),
            scratch_shapes=[
                pltpu.VMEM((RING, B, 1, D), jnp.float32),
                pltpu.SemaphoreType.DMA((RING,)),
                pltpu.SemaphoreType.DMA((RING,)),
            ],
        ),
        out_shape=jax.ShapeDtypeStruct((B, C, D), jnp.float32),
    )
    return f(perm32, x3d)


def kernel(inputs, permutation):
    x3d = inputs.reshape(B, C, D)
    perm32 = permutation.astype(jnp.int32)
    return _shuffle(x3d, perm32).reshape(B, C, H, W)


# speedup vs baseline: 1.0132x; 1.0048x over previous
import jax
import jax.numpy as jnp
from jax.experimental import pallas as pl
from jax.experimental.pallas import tpu as pltpu

B, C, H, W = 16, 384, 64, 64
D = H * W
CG = 16
NG = C // CG
RINGG = 6
GLAG = 3


def _dma_body(idx_ref, x_hbm, out_hbm, buf, gsems, ssems):
    pending_in = [None] * RINGG
    pending_out = [None] * RINGG

    def start_in(g):
        slot = g % RINGG
        ds = []
        for j in range(CG):
            d = pltpu.make_async_copy(
                x_hbm.at[:, pl.ds(idx_ref[CG * g + j], 1), :],
                buf.at[slot, :, pl.ds(j, 1), :],
                gsems.at[slot],
            )
            d.start()
            ds.append(d)
        pending_in[slot] = ds

    def start_out(g):
        slot = g % RINGG
        for d in pending_in[slot]:
            d.wait()
        ds = []
        for b in range(B):
            d = pltpu.make_async_copy(
                buf.at[slot, pl.ds(b, 1), :, :],
                out_hbm.at[pl.ds(b, 1), pl.ds(CG * g, CG), :],
                ssems.at[slot],
            )
            d.start()
            ds.append(d)
        pending_out[slot] = ds

    for g in range(NG + GLAG):
        if g < NG:
            slot = g % RINGG
            if pending_out[slot] is not None:
                for d in pending_out[slot]:
                    d.wait()
                pending_out[slot] = None
            start_in(g)
        if g >= GLAG:
            start_out(g - GLAG)
    for slot in range(RINGG):
        if pending_out[slot] is not None:
            for d in pending_out[slot]:
                d.wait()


@jax.jit
def _shuffle(x3d, perm32):
    f = pl.pallas_call(
        _dma_body,
        grid_spec=pltpu.PrefetchScalarGridSpec(
            num_scalar_prefetch=1,
            grid=(1,),
            in_specs=[pl.BlockSpec(memory_space=pltpu.HBM)],
            out_specs=pl.BlockSpec(memory_space=pltpu.HBM),
            scratch_shapes=[
                pltpu.VMEM((RINGG, B, CG, D), jnp.float32),
                pltpu.SemaphoreType.DMA((RINGG,)),
                pltpu.SemaphoreType.DMA((RINGG,)),
            ],
        ),
        out_shape=jax.ShapeDtypeStruct((B, C, D), jnp.float32),
    )
    return f(perm32, x3d)


def kernel(inputs, permutation):
    x3d = inputs.reshape(B, C, D)
    perm32 = permutation.astype(jnp.int32)
    return _shuffle(x3d, perm32).reshape(B, C, H, W)


# speedup vs baseline: 1.0136x; 1.0005x over previous
import jax
import jax.numpy as jnp
from jax.experimental import pallas as pl
from jax.experimental.pallas import tpu as pltpu

B, C, H, W = 16, 384, 64, 64
D = H * W
CG = 32
NG = C // CG
RINGG = 4
GLAG = 2


def _dma_body(idx_ref, x_hbm, out_hbm, buf, gsems, ssems):
    pending_in = [None] * RINGG
    pending_out = [None] * RINGG

    def start_in(g):
        slot = g % RINGG
        ds = []
        for j in range(CG):
            d = pltpu.make_async_copy(
                x_hbm.at[:, pl.ds(idx_ref[CG * g + j], 1), :],
                buf.at[slot, :, pl.ds(j, 1), :],
                gsems.at[slot],
            )
            d.start()
            ds.append(d)
        pending_in[slot] = ds

    def start_out(g):
        slot = g % RINGG
        for d in pending_in[slot]:
            d.wait()
        ds = []
        for b in range(B):
            d = pltpu.make_async_copy(
                buf.at[slot, pl.ds(b, 1), :, :],
                out_hbm.at[pl.ds(b, 1), pl.ds(CG * g, CG), :],
                ssems.at[slot],
            )
            d.start()
            ds.append(d)
        pending_out[slot] = ds

    for g in range(NG + GLAG):
        if g < NG:
            slot = g % RINGG
            if pending_out[slot] is not None:
                for d in pending_out[slot]:
                    d.wait()
                pending_out[slot] = None
            start_in(g)
        if g >= GLAG:
            start_out(g - GLAG)
    for slot in range(RINGG):
        if pending_out[slot] is not None:
            for d in pending_out[slot]:
                d.wait()


@jax.jit
def _shuffle(x3d, perm32):
    f = pl.pallas_call(
        _dma_body,
        grid_spec=pltpu.PrefetchScalarGridSpec(
            num_scalar_prefetch=1,
            grid=(1,),
            in_specs=[pl.BlockSpec(memory_space=pltpu.HBM)],
            out_specs=pl.BlockSpec(memory_space=pltpu.HBM),
            scratch_shapes=[
                pltpu.VMEM((RINGG, B, CG, D), jnp.float32),
                pltpu.SemaphoreType.DMA((RINGG,)),
                pltpu.SemaphoreType.DMA((RINGG,)),
            ],
        ),
        out_shape=jax.ShapeDtypeStruct((B, C, D), jnp.float32),
    )
    return f(perm32, x3d)


def kernel(inputs, permutation):
    x3d = inputs.reshape(B, C, D)
    perm32 = permutation.astype(jnp.int32)
    return _shuffle(x3d, perm32).reshape(B, C, H, W)
